# baseline (speedup 1.0000x reference)
import jax
import jax.numpy as jnp
from jax.experimental import pallas as pl
from jax.experimental.pallas import tpu as pltpu

_K_SPARSITY = 64
_R_BLK = 256
_C_BLK = 256


def kernel(x, W):
    B, K = x.shape
    N, K2 = W.shape
    assert K == K2 and B % _R_BLK == 0 and N % _C_BLK == 0
    n_row = B // _R_BLK
    nc = N // _C_BLK
    bits_per_step = -(-32 // nc)

    def body(x_ref, w_ref, out_ref, acc_a, acc_b, t_ref):
        j = pl.program_id(1)

        s_blk = jax.lax.dot_general(
            x_ref[...], w_ref[...],
            dimension_numbers=(((1,), (1,)), ((), ())),
            preferred_element_type=jnp.float32,
        )
        off = pl.multiple_of(j * _C_BLK, _C_BLK)
        acc_a[:, pl.ds(off, _C_BLK)] = s_blk

        @pl.when(j == 0)
        def _to_keys():
            bits = jax.lax.bitcast_convert_type(acc_b[...], jnp.int32)
            key = jnp.where(bits < 0, bits ^ jnp.int32(0x7FFFFFFF), bits)
            acc_b[...] = jax.lax.bitcast_convert_type(key, jnp.float32)
            t_ref[...] = jnp.full(t_ref.shape, jnp.iinfo(jnp.int32).min,
                                  jnp.int32)

        for db in range(bits_per_step):
            shift = jnp.maximum(31 - (bits_per_step * j + db), 0)
            add = jnp.left_shift(jnp.int32(1), shift)
            key = jax.lax.bitcast_convert_type(acc_b[...], jnp.int32)
            cand = t_ref[...] + add
            cnt = jnp.sum((key >= cand).astype(jnp.int32), axis=1,
                          keepdims=True)
            t_ref[...] = jnp.where(cnt >= _K_SPARSITY, cand, t_ref[...])

        @pl.when(j == nc - 1)
        def _mask_write_and_swap():
            key = jax.lax.bitcast_convert_type(acc_b[...], jnp.int32)
            s = jax.lax.bitcast_convert_type(
                jnp.where(key < 0, key ^ jnp.int32(0x7FFFFFFF), key),
                jnp.float32)
            mask = (key >= t_ref[...]) & (s > 0)
            out_ref[...] = jnp.where(mask, s, 0.0)
            acc_b[...] = acc_a[...]

    return pl.pallas_call(
        body,
        grid=(n_row + 1, nc),
        in_specs=[
            pl.BlockSpec((_R_BLK, K),
                         lambda i, j: (jnp.minimum(i, n_row - 1), 0)),
            pl.BlockSpec((_C_BLK, K), lambda i, j: (j, 0)),
        ],
        out_specs=pl.BlockSpec((_R_BLK, N),
                               lambda i, j: (jnp.maximum(i - 1, 0), 0)),
        out_shape=jax.ShapeDtypeStruct((B, N), jnp.float32),
        scratch_shapes=[
            pltpu.VMEM((_R_BLK, N), jnp.float32),
            pltpu.VMEM((_R_BLK, N), jnp.float32),
            pltpu.VMEM((_R_BLK, 1), jnp.int32),
        ],
        compiler_params=pltpu.CompilerParams(
            dimension_semantics=("arbitrary", "arbitrary"),
        ),
    )(x, W)


# speedup vs baseline: 1.0409x; 1.0409x over previous
import jax
import jax.numpy as jnp
from jax.experimental import pallas as pl
from jax.experimental.pallas import tpu as pltpu

_K_SPARSITY = 64
_R_BLK = 256
_C_BLK = 256


def kernel(x, W):
    B, K = x.shape
    N, K2 = W.shape
    assert K == K2 and B % _R_BLK == 0 and N % _C_BLK == 0
    n_row = B // _R_BLK
    nc = N // _C_BLK
    assert nc == 32

    def body(x_ref, w_ref, out_ref, hi_a, lo_a, hi_b, lo_b, mlo_b, t_ref,
             cgt_ref):
        j = pl.program_id(1)

        s_blk = jax.lax.dot_general(
            x_ref[...], w_ref[...],
            dimension_numbers=(((1,), (1,)), ((), ())),
            preferred_element_type=jnp.float32,
        )
        bits = jax.lax.bitcast_convert_type(s_blk, jnp.int32)
        key_blk = jnp.where(bits < 0, bits ^ jnp.int32(0x7FFFFFFF), bits)
        off = pl.multiple_of(j * _C_BLK, _C_BLK)
        hi_a[:, pl.ds(off, _C_BLK)] = (key_blk >> 16).astype(jnp.int16)
        lo_a[:, pl.ds(off, _C_BLK)] = (
            (key_blk & jnp.int32(0xFFFF)) - 32768).astype(jnp.int16)

        @pl.when(j == 0)
        def _t_init():
            t_ref[...] = jnp.full(t_ref.shape, jnp.iinfo(jnp.int32).min,
                                  jnp.int32)

        CH = min(2048, N)
        n_chunk = N // CH

        def _psum16(m16):
            w = m16.shape[1]
            while w > 128:
                w //= 2
                m16 = m16[:, :w] + m16[:, w:]
            return jnp.sum(m16.astype(jnp.int32), axis=1, keepdims=True)

        @pl.when(j == 16)
        def _phase_boundary():
            t_hi = (t_ref[...] >> 16).astype(jnp.int16)
            cgt = jnp.zeros((_R_BLK, 1), jnp.int32)
            for c in range(n_chunk):
                hi_c = hi_b[:, c * CH:(c + 1) * CH]
                cgt = cgt + _psum16((hi_c > t_hi).astype(jnp.int16))
                mlo_b[:, c * CH:(c + 1) * CH] = jnp.where(
                    hi_c == t_hi, lo_b[:, c * CH:(c + 1) * CH],
                    jnp.int16(-32768))
            cgt_ref[...] = cgt

        shift = 31 - j
        add = jnp.left_shift(jnp.int32(1), shift)
        cand = t_ref[...] + add

        def _count_hi():
            ch = (cand >> 16).astype(jnp.int16)
            cnt = jnp.zeros((_R_BLK, 1), jnp.int32)
            for c in range(n_chunk):
                cnt = cnt + _psum16(
                    (hi_b[:, c * CH:(c + 1) * CH] >= ch).astype(jnp.int16))
            return cnt

        def _count_lo():
            clb = ((cand & jnp.int32(0xFFFF)) - 32768).astype(jnp.int16)
            cnt = cgt_ref[...]
            for c in range(n_chunk):
                cnt = cnt + _psum16(
                    (mlo_b[:, c * CH:(c + 1) * CH] >= clb).astype(jnp.int16))
            return cnt

        cnt = jax.lax.cond(j < 16, _count_hi, _count_lo)
        t_ref[...] = jnp.where(cnt >= _K_SPARSITY, cand, t_ref[...])

        @pl.when(j == nc - 1)
        def _mask_write_and_swap():
            t = t_ref[...]
            for c in range(n_chunk):
                sl = slice(c * CH, (c + 1) * CH)
                key = jnp.left_shift(hi_b[:, sl].astype(jnp.int32), 16) | (
                    lo_b[:, sl].astype(jnp.int32) + 32768)
                s = jax.lax.bitcast_convert_type(
                    jnp.where(key < 0, key ^ jnp.int32(0x7FFFFFFF), key),
                    jnp.float32)
                mask = (key >= t) & (s > 0)
                out_ref[:, sl] = jnp.where(mask, s, 0.0)
            for c in range(n_chunk):
                sl = slice(c * CH, (c + 1) * CH)
                hi_b[:, sl] = hi_a[:, sl]
                lo_b[:, sl] = lo_a[:, sl]

    return pl.pallas_call(
        body,
        grid=(n_row + 1, nc),
        in_specs=[
            pl.BlockSpec((_R_BLK, K),
                         lambda i, j: (jnp.minimum(i, n_row - 1), 0)),
            pl.BlockSpec((_C_BLK, K), lambda i, j: (j, 0)),
        ],
        out_specs=pl.BlockSpec((_R_BLK, N),
                               lambda i, j: (jnp.maximum(i - 1, 0), 0)),
        out_shape=jax.ShapeDtypeStruct((B, N), jnp.float32),
        scratch_shapes=[
            pltpu.VMEM((_R_BLK, N), jnp.int16),
            pltpu.VMEM((_R_BLK, N), jnp.int16),
            pltpu.VMEM((_R_BLK, N), jnp.int16),
            pltpu.VMEM((_R_BLK, N), jnp.int16),
            pltpu.VMEM((_R_BLK, N), jnp.int16),
            pltpu.VMEM((_R_BLK, 1), jnp.int32),
            pltpu.VMEM((_R_BLK, 1), jnp.int32),
        ],
        compiler_params=pltpu.CompilerParams(
            dimension_semantics=("arbitrary", "arbitrary"),
        ),
    )(x, W)
